# baseline (speedup 1.0000x reference)
import functools

import jax
import jax.numpy as jnp
from jax import lax
from jax.experimental import pallas as pl
from jax.experimental.pallas import tpu as pltpu
from jax.experimental.pallas import tpu_sc as plsc

_P = 0.5
_NC = 2
_NS = 16


def _mlp_body(x_ref, w_ref, b_ref, o_ref):
    o_ref[...] = jnp.tanh(
        jnp.dot(x_ref[...], w_ref[...], preferred_element_type=jnp.float32)
        + b_ref[...]
    )


def _out_body(x_ref, a_ref, dn_ref, w_ref, b_ref, o_ref):
    x = x_ref[...]
    mean_h = a_ref[0] * dn_ref[...]
    hr = jnp.where(dn_ref[...] > 0.0, mean_h, x)
    o_ref[...] = (
        jnp.dot(x + hr, w_ref[...], preferred_element_type=jnp.float32)
        + b_ref[...]
    )


def _make_agg(a_rows, d_in, e):
    e_per_t = e // _NS
    k = 400
    n_iter = e_per_t // k
    rows_per_tile = a_rows // _NS
    mesh = plsc.VectorSubcoreMesh(core_axis_name="c", subcore_axis_name="s")

    @functools.partial(
        pl.kernel,
        mesh=mesh,
        out_type=jax.ShapeDtypeStruct((_NC, a_rows, d_in), jnp.float32),
        scratch_types=[
            pltpu.VMEM((k,), jnp.int32),
            pltpu.VMEM((k,), jnp.int32),
            pltpu.VMEM((k, d_in), jnp.float32),
            pltpu.VMEM_SHARED((a_rows, d_in), jnp.float32),
            pltpu.SemaphoreType.DMA,
        ],
    )
    def agg(x_hbm, gidx_hbm, dst0_hbm, dst1_hbm, zero_hbm, out_hbm,
            idx_v, dst_v, rows_v, acc_sh, sem):
        cid = lax.axis_index("c")
        sid = lax.axis_index("s")
        r0 = sid * rows_per_tile
        pltpu.sync_copy(zero_hbm.at[pl.ds(r0, rows_per_tile)],
                        acc_sh.at[pl.ds(r0, rows_per_tile)])
        plsc.subcore_barrier()

        e0 = sid * e_per_t

        def run(dst_hbm):
            def body(g, carry):
                base = e0 + g * k
                pltpu.sync_copy(gidx_hbm.at[pl.ds(base, k)], idx_v)
                pltpu.sync_copy(dst_hbm.at[pl.ds(base, k)], dst_v)
                pltpu.async_copy(x_hbm.at[idx_v], rows_v, sem).wait()
                pltpu.sync_copy(rows_v, acc_sh.at[dst_v], add=True)
                return carry

            lax.fori_loop(0, n_iter, body, 0)

        pl.when(cid == 0)(lambda: run(dst0_hbm))
        pl.when(cid == 1)(lambda: run(dst1_hbm))
        plsc.subcore_barrier()
        pltpu.sync_copy(acc_sh.at[pl.ds(r0, rows_per_tile)],
                        out_hbm.at[cid, pl.ds(r0, rows_per_tile)])

    return agg


def kernel(x, edge_index, W_mlp, b_mlp, W_lin, b_lin):
    n, d_in = x.shape
    c = W_mlp.shape[0]
    d_out = W_lin.shape[0]
    e = edge_index.shape[1]
    src = edge_index[0]
    dst = edge_index[1]

    bm = 1000
    grid = (n // bm,)

    t = pl.pallas_call(
        _mlp_body,
        grid=grid,
        in_specs=[
            pl.BlockSpec((bm, d_in), lambda i: (i, 0)),
            pl.BlockSpec((d_in, c), lambda i: (0, 0)),
            pl.BlockSpec((1, c), lambda i: (0, 0)),
        ],
        out_specs=pl.BlockSpec((bm, c), lambda i: (i, 0)),
        out_shape=jax.ShapeDtypeStruct((n, c), jnp.float32),
    )(x, W_mlp.T, b_mlp[None])

    d = jnp.sum(jnp.abs(t[src] - t[dst]), axis=1)

    deg = jnp.bincount(dst, length=n)
    num_keep = jnp.ceil(_P * deg.astype(jnp.float32)).astype(jnp.int32)
    order = jnp.lexsort((d, dst))
    dst_s = dst[order]
    src_s = src[order]
    start = jnp.cumsum(deg) - deg
    rank = jnp.arange(e, dtype=jnp.int32) - start[dst_s].astype(jnp.int32)
    keep = rank < num_keep[dst_s]

    split = 5120
    a_rows = 5248
    trash = jnp.int32(a_rows - 8)
    x_pad = jnp.zeros((n + 8, d_in), jnp.float32).at[:n].set(x)
    gidx = jnp.where(keep, src_s, jnp.int32(n))
    dst0 = jnp.where(dst_s < split, dst_s, trash)
    dst1 = jnp.where(dst_s >= split, dst_s - split, trash)
    zero = jnp.zeros((a_rows, d_in), jnp.float32)

    acc2 = _make_agg(a_rows, d_in, e)(x_pad, gidx, dst0, dst1, zero)

    inv_denom = jnp.where(
        deg > 0, 1.0 / jnp.maximum(num_keep, 1).astype(jnp.float32), 0.0
    )[:, None]

    bo = 512
    nb_half = split // bo
    out = pl.pallas_call(
        _out_body,
        grid=((n + bo - 1) // bo,),
        in_specs=[
            pl.BlockSpec((bo, d_in), lambda i: (i, 0)),
            pl.BlockSpec((1, bo, d_in),
                         lambda i: (i // nb_half, i % nb_half, 0)),
            pl.BlockSpec((bo, 1), lambda i: (i, 0)),
            pl.BlockSpec((d_in, d_out), lambda i: (0, 0)),
            pl.BlockSpec((1, d_out), lambda i: (0, 0)),
        ],
        out_specs=pl.BlockSpec((bo, d_out), lambda i: (i, 0)),
        out_shape=jax.ShapeDtypeStruct((n, d_out), jnp.float32),
    )(x, acc2, inv_denom, W_lin.T, b_lin[None])

    return out
